# baseline (speedup 1.0000x reference)
import functools

import jax
import jax.numpy as jnp
from jax import lax
from jax.experimental import pallas as pl
from jax.experimental.pallas import tpu as pltpu
from jax.experimental.pallas import tpu_sc as plsc

B, T, D = 4096, 200, 64
ROWS = B * T
NC, NS = 2, 16
NW = NC * NS
ROWS_PER_W = ROWS // NW
GATHER = 100
CHUNK = 2 * T
G_PER_CHUNK = CHUNK // GATHER
NCH = ROWS_PER_W // CHUNK
IDX_ROWS = ROWS // GATHER

_mesh = plsc.VectorSubcoreMesh(core_axis_name="c", subcore_axis_name="s")


@functools.partial(
    pl.kernel,
    out_type=jax.ShapeDtypeStruct((ROWS, D), jnp.float32),
    mesh=_mesh,
    scratch_types=[
        pltpu.VMEM((G_PER_CHUNK, GATHER), jnp.int32),
        pltpu.VMEM((CHUNK, D), jnp.float32),
        pltpu.VMEM((T, D), jnp.float32),
        pltpu.SemaphoreType.DMA,
    ],
    compiler_params=pltpu.CompilerParams(use_tc_tiling_on_sc=False),
)
def _sc_embed(idx_hbm, table_hbm, pos_hbm, out_hbm, idx_v, rows_v, pos_v, sem):
    wid = lax.axis_index("s") * NC + lax.axis_index("c")
    pltpu.sync_copy(pos_hbm, pos_v)

    def chunk_body(c, carry):
        ibase = wid * (ROWS_PER_W // GATHER) + c * G_PER_CHUNK
        pltpu.sync_copy(idx_hbm.at[pl.ds(ibase, G_PER_CHUNK)], idx_v)
        copies = [
            pltpu.async_copy(
                table_hbm.at[idx_v.at[j]],
                rows_v.at[pl.ds(j * GATHER, GATHER)],
                sem,
            )
            for j in range(G_PER_CHUNK)
        ]
        for cp in copies:
            cp.wait()

        def add_body(r, _):
            for s2 in range(CHUNK // T):
                for dd in range(D // 16):
                    sl = pl.ds(dd * 16, 16)
                    rows_v[s2 * T + r, sl] = rows_v[s2 * T + r, sl] + pos_v[r, sl]
            return _

        lax.fori_loop(0, T, add_body, 0, unroll=2)

        obase = wid * ROWS_PER_W + c * CHUNK
        pltpu.sync_copy(rows_v, out_hbm.at[pl.ds(obase, CHUNK)])
        return carry

    lax.fori_loop(0, NCH, chunk_body, 0)


def kernel(x, token_table, pos_table):
    idx = x.astype(jnp.int32).reshape(IDX_ROWS, GATHER)
    out = _sc_embed(idx, token_table, pos_table)
    return out.reshape(B, T, D)


# speedup vs baseline: 1.4291x; 1.4291x over previous
import functools

import jax
import jax.numpy as jnp
from jax import lax
from jax.experimental import pallas as pl
from jax.experimental.pallas import tpu as pltpu
from jax.experimental.pallas import tpu_sc as plsc

B, T, D = 4096, 200, 64
ROWS = B * T
NC, NS = 2, 16
NW = NC * NS
ROWS_PER_W = ROWS // NW
GATHER = 100
CHUNK = T
G_PER_CHUNK = CHUNK // GATHER
NCH = ROWS_PER_W // CHUNK
IDX_ROWS = ROWS // GATHER
IDX_PER_W = ROWS_PER_W // GATHER
NBUF = 4

_mesh = plsc.VectorSubcoreMesh(core_axis_name="c", subcore_axis_name="s")


@functools.partial(
    pl.kernel,
    out_type=jax.ShapeDtypeStruct((ROWS, D), jnp.float32),
    mesh=_mesh,
    scratch_types=[
        pltpu.VMEM((IDX_PER_W, GATHER), jnp.int32),
        pltpu.VMEM((NBUF, CHUNK, D), jnp.float32),
        pltpu.VMEM((T, D), jnp.float32),
        pltpu.SemaphoreType.DMA((NBUF,)),
        pltpu.SemaphoreType.DMA((NBUF,)),
    ],
    compiler_params=pltpu.CompilerParams(use_tc_tiling_on_sc=False),
)
def _sc_embed(idx_hbm, table_hbm, pos_hbm, out_hbm, idx_v, rows_v, pos_v,
              g_sem, w_sem):
    wid = lax.axis_index("s") * NC + lax.axis_index("c")
    pltpu.sync_copy(pos_hbm, pos_v)
    pltpu.sync_copy(idx_hbm.at[pl.ds(wid * IDX_PER_W, IDX_PER_W)], idx_v)
    out_base = wid * ROWS_PER_W

    def issue_gather(c, b):
        for j in range(G_PER_CHUNK):
            pltpu.async_copy(
                table_hbm.at[idx_v.at[c * G_PER_CHUNK + j]],
                rows_v.at[b, pl.ds(j * GATHER, GATHER)],
                g_sem.at[b],
            )

    def wait_gathers(b):
        pltpu.make_async_copy(
            out_hbm.at[pl.ds(0, CHUNK)], rows_v.at[b], g_sem.at[b]
        ).wait()

    def add_pos(b):
        @plsc.parallel_loop(0, T, 1, unroll=4)
        def _(r):
            for dd in range(D // 16):
                sl = pl.ds(dd * 16, 16)
                rows_v[b, r, sl] = rows_v[b, r, sl] + pos_v[r, sl]

    def issue_wb(c, b):
        pltpu.async_copy(
            rows_v.at[b], out_hbm.at[pl.ds(out_base + c * CHUNK, CHUNK)],
            w_sem.at[b],
        )

    def wait_wb(b):
        pltpu.make_async_copy(
            rows_v.at[b], out_hbm.at[pl.ds(0, CHUNK)], w_sem.at[b]
        ).wait()

    def finish(c, b):
        wait_gathers(b)
        add_pos(b)
        issue_wb(c, b)

    for c in range(NBUF - 1):
        issue_gather(c, c)
    finish(0, 0)
    issue_gather(NBUF - 1, NBUF - 1)
    for c in range(1, NBUF):
        finish(c, c % NBUF)
        wait_wb((c - 1) % NBUF)
        issue_gather(c + NBUF - 1, (c - 1) % NBUF)

    def outer(i, carry):
        for b2 in range(NBUF):
            c = i * NBUF + b2
            finish(c, b2)
            wait_wb((b2 + NBUF - 1) % NBUF)
            issue_gather(c + NBUF - 1, (b2 + NBUF - 1) % NBUF)
        return carry

    lax.fori_loop(1, NCH // NBUF - 1, outer, 0)

    c0 = NCH - NBUF
    finish(c0, c0 % NBUF)
    wait_wb((c0 - 1) % NBUF)
    issue_gather(NCH - 1, (c0 - 1) % NBUF)
    for c in range(c0 + 1, NCH):
        finish(c, c % NBUF)
    for b in range(NBUF):
        wait_wb(b)


def kernel(x, token_table, pos_table):
    idx = x.astype(jnp.int32).reshape(IDX_ROWS, GATHER)
    out = _sc_embed(idx, token_table, pos_table)
    return out.reshape(B, T, D)
